# baseline (device time: 326345 ns/iter reference)
import jax
import jax.numpy as jnp
from jax import lax
from jax.experimental import pallas as pl
from jax.experimental.pallas import tpu as pltpu

N_DEV = 4
M = 4096
K = 1024
N_OUT = 2048
CH = M // N_DEV
HALF = N_OUT // 2
HCH = CH // 2


def _body(x_ref, w_ref, o_ref, w_v, x_v, part, recv,
          sem_x, rs_send, rs_recv, ag_send, ag_recv, st_sems):
    i = lax.axis_index("i")
    right = lax.rem(i + 1, N_DEV)
    left = lax.rem(i + N_DEV - 1, N_DEV)

    bar = pltpu.get_barrier_semaphore()
    for nbr in (left, right):
        pl.semaphore_signal(bar, inc=1, device_id=(nbr,),
                            device_id_type=pl.DeviceIdType.MESH)
    pl.semaphore_wait(bar, 2)

    def cmod(v):
        return lax.rem(v + 2 * N_DEV, N_DEV)

    dirs = ((0, 1, right), (1, -1, left))

    chunks = [cmod(i - 1), cmod(i + 1), cmod(i + 2), i]
    x_cps = [
        pltpu.make_async_copy(
            x_ref.at[pl.ds(chunks[k] * CH, CH), :], x_v.at[k % 2],
            sem_x.at[k % 2])
        for k in range(4)
    ]

    w_cps = [
        pltpu.make_async_copy(
            w_ref.at[:, pl.ds(h * HALF, HALF)], x_v.at[h], sem_x.at[h])
        for h in range(2)
    ]
    for cp in w_cps:
        cp.start()
    for h, cp in enumerate(w_cps):
        cp.wait()
        w_v[:, h * HALF:(h + 1) * HALF] = x_v[h].astype(jnp.bfloat16)

    x_cps[0].start()
    x_cps[0].wait()
    x_cps[1].start()

    def gemm(k, slot):
        part[slot] = jnp.dot(
            x_v[k % 2].astype(jnp.bfloat16), w_v[...],
            preferred_element_type=jnp.float32)

    gemm(0, 0)
    x_cps[1].wait()
    x_cps[2].start()
    gemm(1, 1)

    rs0 = []
    for d, sgn, tgt in dirs:
        rdma = pltpu.make_async_remote_copy(
            src_ref=part.at[d, pl.ds(d * HCH, HCH), :],
            dst_ref=recv.at[d, 0],
            send_sem=rs_send.at[d, 0], recv_sem=rs_recv.at[d, 0],
            device_id=(tgt,), device_id_type=pl.DeviceIdType.MESH)
        rdma.start()
        rs0.append(rdma)

    x_cps[2].wait()
    x_cps[3].start()
    gemm(2, 2)

    for (d, sgn, tgt), rdma in zip(dirs, rs0):
        rdma.wait_recv()
        recv[d, 0] = recv[d, 0] + part[2, d * HCH:(d + 1) * HCH, :]
    for rdma in rs0:
        rdma.wait_send()

    rs1 = []
    for d, sgn, tgt in dirs:
        rdma = pltpu.make_async_remote_copy(
            src_ref=recv.at[d, 0], dst_ref=recv.at[d, 1],
            send_sem=rs_send.at[d, 1], recv_sem=rs_recv.at[d, 1],
            device_id=(tgt,), device_id_type=pl.DeviceIdType.MESH)
        rdma.start()
        rs1.append(rdma)

    x_cps[3].wait()
    gemm(3, 2)

    adds1 = (part.at[1, 0:HCH, :], part.at[0, HCH:CH, :])
    for (d, sgn, tgt), rdma in zip(dirs, rs1):
        rdma.wait_recv()
        recv[d, 1] = recv[d, 1] + adds1[d][...]
    for rdma in rs1:
        rdma.wait_send()

    rs2 = []
    for d, sgn, tgt in dirs:
        rdma = pltpu.make_async_remote_copy(
            src_ref=recv.at[d, 1], dst_ref=recv.at[d, 0],
            send_sem=rs_send.at[d, 0], recv_sem=rs_recv.at[d, 0],
            device_id=(tgt,), device_id_type=pl.DeviceIdType.MESH)
        rdma.start()
        rs2.append(rdma)
    for (d, sgn, tgt), rdma in zip(dirs, rs2):
        rdma.wait_recv()
        recv[d, 0] = jax.nn.silu(
            recv[d, 0] + part[2, d * HCH:(d + 1) * HCH, :])
    for rdma in rs2:
        rdma.wait_send()

    sts = []
    for d, _, _ in dirs:
        st = pltpu.make_async_copy(
            recv.at[d, 0],
            o_ref.at[pl.ds(i * CH + d * HCH, HCH), :],
            st_sems.at[d])
        st.start()
        sts.append(st)

    for s in range(N_DEV - 1):
        rdmas = []
        for d, sgn, tgt in dirs:
            c_send = cmod(i - sgn * s)
            dst = o_ref.at[pl.ds(c_send * CH + d * HCH, HCH), :]
            src = recv.at[d, 0] if s == 0 else dst
            rdma = pltpu.make_async_remote_copy(
                src_ref=src, dst_ref=dst,
                send_sem=ag_send.at[d, s], recv_sem=ag_recv.at[d, s],
                device_id=(tgt,), device_id_type=pl.DeviceIdType.MESH)
            rdma.start()
            rdmas.append(rdma)
        for rdma in rdmas:
            rdma.wait_recv()
        for rdma in rdmas:
            rdma.wait_send()
    for st in sts:
        st.wait()


def kernel(x, w_mat):
    return pl.pallas_call(
        _body,
        out_shape=jax.ShapeDtypeStruct((M, N_OUT), jnp.float32),
        in_specs=[pl.BlockSpec(memory_space=pl.ANY),
                  pl.BlockSpec(memory_space=pl.ANY)],
        out_specs=pl.BlockSpec(memory_space=pl.ANY),
        scratch_shapes=[
            pltpu.VMEM((K, N_OUT), jnp.bfloat16),
            pltpu.VMEM((2, CH, K), jnp.float32),
            pltpu.VMEM((3, CH, N_OUT), jnp.float32),
            pltpu.VMEM((2, 2, HCH, N_OUT), jnp.float32),
            pltpu.SemaphoreType.DMA((2,)),
            pltpu.SemaphoreType.DMA((2, 2)),
            pltpu.SemaphoreType.DMA((2, 2)),
            pltpu.SemaphoreType.DMA((2, N_DEV - 1)),
            pltpu.SemaphoreType.DMA((2, N_DEV - 1)),
            pltpu.SemaphoreType.DMA((2,)),
        ],
        compiler_params=pltpu.CompilerParams(
            collective_id=0,
            vmem_limit_bytes=64 * 1024 * 1024,
        ),
    )(x, w_mat)


# device time: 307297 ns/iter; 1.0620x vs baseline; 1.0620x over previous
import jax
import jax.numpy as jnp
from jax import lax
from jax.experimental import pallas as pl
from jax.experimental.pallas import tpu as pltpu

N_DEV = 4
M = 4096
K = 1024
N_OUT = 2048
CH = M // N_DEV
HCH = CH // 2
QR = HCH // 2


def _body(x_ref, w_ref, o_ref, w_v, x_v, part, recv,
          sem_w, sem_x, rs_send, rs_recv, ag_send, ag_recv, st_sems):
    i = lax.axis_index("i")
    right = lax.rem(i + 1, N_DEV)
    left = lax.rem(i + N_DEV - 1, N_DEV)

    bar = pltpu.get_barrier_semaphore()
    for nbr in (left, right):
        pl.semaphore_signal(bar, inc=1, device_id=(nbr,),
                            device_id_type=pl.DeviceIdType.MESH)
    pl.semaphore_wait(bar, 2)

    def cmod(v):
        return lax.rem(v + 2 * N_DEV, N_DEV)

    dirs = ((0, 1, right), (1, -1, left))
    c1, c2, c3 = cmod(i - 1), cmod(i + 1), cmod(i + 2)

    blocks = [(c1, 0, 0), (c2, 1, 1), (c3, 0, 2), (c3, 1, 2),
              (c2, 0, 1), (c1, 1, 0), (i, 0, 2), (i, 1, 2)]
    x_cps = [
        pltpu.make_async_copy(
            x_ref.at[pl.ds(c * CH + h * HCH, HCH), :], x_v.at[k % 2],
            sem_x.at[k % 2])
        for k, (c, h, _) in enumerate(blocks)
    ]

    def gemm(k):
        _, h, slot = blocks[k]
        part[slot, h * HCH:(h + 1) * HCH, :] = jnp.dot(
            x_v[k % 2].astype(jnp.bfloat16), w_v[...],
            preferred_element_type=jnp.float32)

    w_cp = pltpu.make_async_copy(w_ref, part.at[2], sem_w)
    w_cp.start()
    x_cps[0].start()
    w_cp.wait()
    w_v[...] = part[2].astype(jnp.bfloat16)

    x_cps[0].wait()
    x_cps[1].start()
    gemm(0)
    x_cps[1].wait()
    x_cps[2].start()
    gemm(1)

    def rs_rdma(d, tgt, s, q, src):
        slot = s % 2
        return pltpu.make_async_remote_copy(
            src_ref=(part.at[d, pl.ds(d * HCH + q * QR, QR), :] if src is part
                     else src.at[d, (s - 1) % 2, pl.ds(q * QR, QR), :]),
            dst_ref=recv.at[d, slot, pl.ds(q * QR, QR), :],
            send_sem=rs_send.at[d, slot, q], recv_sem=rs_recv.at[d, slot, q],
            device_id=(tgt,), device_id_type=pl.DeviceIdType.MESH)

    rs = {}
    for d, sgn, tgt in dirs:
        for q in range(2):
            r = rs_rdma(d, tgt, 0, q, part)
            r.start()
            rs[(0, d, q)] = r

    x_cps[2].wait()
    x_cps[3].start()
    gemm(2)
    x_cps[3].wait()
    x_cps[4].start()
    gemm(3)
    x_cps[4].wait()
    x_cps[5].start()
    gemm(4)
    x_cps[5].wait()
    gemm(5)

    add0 = {d: part.at[2, pl.ds(d * HCH, HCH), :] for d in range(2)}
    for q in range(2):
        for d, sgn, tgt in dirs:
            rs[(0, d, q)].wait_recv()
            recv[d, 0, q * QR:(q + 1) * QR, :] = (
                recv[d, 0, q * QR:(q + 1) * QR, :]
                + part[2, d * HCH + q * QR:d * HCH + (q + 1) * QR, :])
            r = rs_rdma(d, tgt, 1, q, recv)
            r.start()
            rs[(1, d, q)] = r

    x_cps[6].start()
    x_cps[6].wait()
    x_cps[7].start()
    gemm(6)
    x_cps[7].wait()
    gemm(7)

    adds1 = (part.at[1, 0:HCH, :], part.at[0, HCH:CH, :])
    for q in range(2):
        for d, sgn, tgt in dirs:
            rs[(1, d, q)].wait_recv()
            recv[d, 1, q * QR:(q + 1) * QR, :] = (
                recv[d, 1, q * QR:(q + 1) * QR, :]
                + adds1[d][q * QR:(q + 1) * QR, :])
            rs[(0, d, q)].wait_send()
            r = rs_rdma(d, tgt, 2, q, recv)
            r.start()
            rs[(2, d, q)] = r

    def ag_rdma(d, tgt, s, q, from_vmem=False):
        c_send = cmod(i - (1 if d == 0 else -1) * s)
        dst = o_ref.at[pl.ds(c_send * CH + d * HCH + q * QR, QR), :]
        src = recv.at[d, 0, pl.ds(q * QR, QR), :] if from_vmem else dst
        return pltpu.make_async_remote_copy(
            src_ref=src, dst_ref=dst,
            send_sem=ag_send.at[d, s, q], recv_sem=ag_recv.at[d, s, q],
            device_id=(tgt,), device_id_type=pl.DeviceIdType.MESH)

    ag = {}
    for q in range(2):
        for d, sgn, tgt in dirs:
            rs[(2, d, q)].wait_recv()
            recv[d, 0, q * QR:(q + 1) * QR, :] = jax.nn.silu(
                recv[d, 0, q * QR:(q + 1) * QR, :]
                + part[2, d * HCH + q * QR:d * HCH + (q + 1) * QR, :])
            r = ag_rdma(d, tgt, 0, q, from_vmem=True)
            r.start()
            ag[(0, d, q)] = r

    sts = []
    for d, _, _ in dirs:
        st = pltpu.make_async_copy(
            recv.at[d, 0],
            o_ref.at[pl.ds(i * CH + d * HCH, HCH), :],
            st_sems.at[d])
        st.start()
        sts.append(st)

    for s in range(N_DEV - 1):
        for q in range(2):
            for d, sgn, tgt in dirs:
                ag[(s, d, q)].wait_recv()
                if s < N_DEV - 2:
                    r = ag_rdma(d, tgt, s + 1, q)
                    r.start()
                    ag[(s + 1, d, q)] = r

    for q in range(2):
        for d, _, _ in dirs:
            rs[(1, d, q)].wait_send()
            rs[(2, d, q)].wait_send()
            for s in range(N_DEV - 1):
                ag[(s, d, q)].wait_send()
    for st in sts:
        st.wait()


def kernel(x, w_mat):
    return pl.pallas_call(
        _body,
        out_shape=jax.ShapeDtypeStruct((M, N_OUT), jnp.float32),
        in_specs=[pl.BlockSpec(memory_space=pl.ANY),
                  pl.BlockSpec(memory_space=pl.ANY)],
        out_specs=pl.BlockSpec(memory_space=pl.ANY),
        scratch_shapes=[
            pltpu.VMEM((K, N_OUT), jnp.bfloat16),
            pltpu.VMEM((2, HCH, K), jnp.float32),
            pltpu.VMEM((3, CH, N_OUT), jnp.float32),
            pltpu.VMEM((2, 2, HCH, N_OUT), jnp.float32),
            pltpu.SemaphoreType.DMA,
            pltpu.SemaphoreType.DMA((2,)),
            pltpu.SemaphoreType.DMA((2, 2, 2)),
            pltpu.SemaphoreType.DMA((2, 2, 2)),
            pltpu.SemaphoreType.DMA((2, N_DEV - 1, 2)),
            pltpu.SemaphoreType.DMA((2, N_DEV - 1, 2)),
            pltpu.SemaphoreType.DMA((2,)),
        ],
        compiler_params=pltpu.CompilerParams(
            collective_id=0,
            vmem_limit_bytes=64 * 1024 * 1024,
        ),
    )(x, w_mat)
